# baseline (speedup 1.0000x reference)
import functools

import jax
import jax.numpy as jnp
from jax import lax
from jax.experimental import pallas as pl
from jax.experimental.pallas import tpu as pltpu
from jax.experimental.pallas import tpu_sc as plsc


_BH = 32


def _tc_kernel(prob_ref, m_ref, idx_ref):
    x = prob_ref[...]
    xt = jnp.swapaxes(x, 1, 2)
    c = xt.shape[1]
    m = jnp.max(xt, axis=1)
    iota = lax.broadcasted_iota(jnp.int32, xt.shape, 1)
    idx = jnp.min(jnp.where(xt == m[:, None, :], iota, c), axis=1)
    m_ref[...] = m
    idx_ref[...] = idx


def _tc_stage(prob, row_base, rows_n):
    hei, wid, cls = prob.shape
    grid = rows_n // _BH
    blk0 = row_base // _BH
    return pl.pallas_call(
        _tc_kernel,
        grid=(grid,),
        in_specs=[pl.BlockSpec((_BH, wid, cls), lambda i: (blk0 + i, 0, 0))],
        out_specs=(
            pl.BlockSpec((_BH, wid), lambda i: (i, 0)),
            pl.BlockSpec((_BH, wid), lambda i: (i, 0)),
        ),
        out_shape=(
            jax.ShapeDtypeStruct((rows_n, wid), jnp.float32),
            jax.ShapeDtypeStruct((rows_n, wid), jnp.int32),
        ),
    )(prob)


def _sc_stage(prob_flat, m_flat, idx_flat, cls, pix_base):
    n = m_flat.shape[0]
    total = prob_flat.shape[0]
    nw = 32
    ppw = n // nw
    rows = ppw // 128
    mesh = plsc.VectorSubcoreMesh(core_axis_name="c", subcore_axis_name="s")

    @functools.partial(
        pl.kernel,
        mesh=mesh,
        out_type=jax.ShapeDtypeStruct((n,), jnp.float32),
        scratch_types=[
            pltpu.VMEM((ppw,), jnp.int32),
            pltpu.VMEM((ppw,), jnp.float32),
            pltpu.VMEM((rows, 128), jnp.int32),
            pltpu.VMEM((rows, 128), jnp.int32),
            pltpu.VMEM((rows, 128), jnp.float32),
            pltpu.VMEM((rows, 128), jnp.float32),
            pltpu.VMEM((ppw,), jnp.float32),
            pltpu.SemaphoreType.DMA,
        ],
    )
    def k(prob_hbm, m_hbm, idx_hbm, out_hbm,
          idx_v, m_v, alo, aup, lo, up, out_v, sem):
        wid = lax.axis_index("s") * 2 + lax.axis_index("c")
        base = wid * ppw
        pltpu.sync_copy(idx_hbm.at[pl.ds(base, ppw)], idx_v)
        pltpu.sync_copy(m_hbm.at[pl.ds(base, ppw)], m_v)
        lane = lax.iota(jnp.int32, 16)

        def tiled_addr(p, c):
            t = (p >> 3) * (cls // 128) + (c >> 7)
            return t * 1024 + (p & 7) * 128 + (c & 127)

        def addr_body(j, carry):
            v = idx_v[pl.ds(j * 16, 16)]
            p = pix_base + base + j * 16 + lane
            vlo = jnp.maximum(v - 1, 0)
            vup = jnp.minimum(v + 1, cls - 1)
            r = j // 8
            cc = (j % 8) * 16
            alo[r, pl.ds(cc, 16)] = tiled_addr(p, vlo)
            aup[r, pl.ds(cc, 16)] = tiled_addr(p, vup)
            return carry

        lax.fori_loop(0, ppw // 16, addr_body, 0)

        wave = 16
        for r0 in range(0, rows, wave):
            handles = []
            for r in range(r0, r0 + wave):
                handles.append(pltpu.async_copy(prob_hbm.at[alo.at[r]], lo.at[r], sem))
                handles.append(pltpu.async_copy(prob_hbm.at[aup.at[r]], up.at[r], sem))
            for h in handles:
                h.wait()

        def comb_body(j, carry):
            r = j // 8
            cc = (j % 8) * 16
            v = idx_v[pl.ds(j * 16, 16)]
            mm = m_v[pl.ds(j * 16, 16)]
            l = jnp.where(v > 0, lo[r, pl.ds(cc, 16)], 0.0)
            u = jnp.where(v < cls - 1, up[r, pl.ds(cc, 16)], 0.0)
            g = jnp.maximum(l, u)
            vf = v.astype(jnp.float32)
            nbr = jnp.where(u > l, vf + 1.0, vf - 1.0)
            fl = (mm * vf + g * nbr) / (mm + g)
            out_v[pl.ds(j * 16, 16)] = fl * jnp.float32(0.035) - jnp.float32(4.0)
            return carry

        lax.fori_loop(0, ppw // 16, comb_body, 0)
        pltpu.sync_copy(out_v, out_hbm.at[pl.ds(base, ppw)])

    return k(prob_flat, m_flat, idx_flat)


_CHUNKS = 4


def kernel(prob):
    hei, wid, cls = prob.shape
    npix = hei * wid
    flat = (
        prob.reshape(npix // 8, 8, cls // 128, 128)
        .transpose(0, 2, 1, 3)
        .reshape(-1)
    )
    ch = hei // _CHUNKS
    disps = []
    for k in range(_CHUNKS):
        mk, idxk = _tc_stage(prob, k * ch, ch)
        disps.append(
            _sc_stage(flat, mk.reshape(-1), idxk.reshape(-1), cls,
                      pix_base=k * ch * wid)
        )
    return jnp.concatenate(disps).reshape(hei, wid)


# speedup vs baseline: 1.0810x; 1.0810x over previous
import functools

import jax
import jax.numpy as jnp
from jax import lax
from jax.experimental import pallas as pl
from jax.experimental.pallas import tpu as pltpu
from jax.experimental.pallas import tpu_sc as plsc


_BH = 32


def _tc_kernel(prob_ref, m_ref, idx_ref, alo_ref, aup_ref):
    x = prob_ref[...]
    xt = jnp.swapaxes(x, 1, 2)
    bh, c, w = xt.shape
    m = jnp.max(xt, axis=1)
    iota = lax.broadcasted_iota(jnp.int32, xt.shape, 1)
    idx = jnp.min(jnp.where(xt == m[:, None, :], iota, c), axis=1)
    m_ref[...] = m
    idx_ref[...] = idx
    p = (
        pl.program_id(0) * (bh * w)
        + lax.broadcasted_iota(jnp.int32, (bh, w), 0) * w
        + lax.broadcasted_iota(jnp.int32, (bh, w), 1)
    )
    pterm = (p >> 3) * (c // 128) * 1024 + (p & 7) * 128

    def tiled_addr(cc):
        return pterm + (cc >> 7) * 1024 + (cc & 127)

    alo_ref[...] = tiled_addr(jnp.maximum(idx - 1, 0))
    aup_ref[...] = tiled_addr(jnp.minimum(idx + 1, c - 1))


def _tc_stage(prob):
    hei, wid, cls = prob.shape
    grid = hei // _BH
    pix_spec = pl.BlockSpec((_BH, wid), lambda i: (i, 0))
    return pl.pallas_call(
        _tc_kernel,
        grid=(grid,),
        in_specs=[pl.BlockSpec((_BH, wid, cls), lambda i: (i, 0, 0))],
        out_specs=(pix_spec, pix_spec, pix_spec, pix_spec),
        out_shape=(
            jax.ShapeDtypeStruct((hei, wid), jnp.float32),
            jax.ShapeDtypeStruct((hei, wid), jnp.int32),
            jax.ShapeDtypeStruct((hei, wid), jnp.int32),
            jax.ShapeDtypeStruct((hei, wid), jnp.int32),
        ),
    )(prob)


def _sc_stage(prob_flat, m_flat, idx_flat, alo_flat, aup_flat, cls):
    n = m_flat.shape[0]
    nw = 32
    ppw = n // nw
    rows = ppw // 128
    mesh = plsc.VectorSubcoreMesh(core_axis_name="c", subcore_axis_name="s")

    @functools.partial(
        pl.kernel,
        mesh=mesh,
        out_type=jax.ShapeDtypeStruct((n,), jnp.float32),
        scratch_types=[
            pltpu.VMEM((ppw,), jnp.int32),
            pltpu.VMEM((ppw,), jnp.float32),
            pltpu.VMEM((rows, 128), jnp.int32),
            pltpu.VMEM((rows, 128), jnp.int32),
            pltpu.VMEM((rows, 128), jnp.float32),
            pltpu.VMEM((rows, 128), jnp.float32),
            pltpu.VMEM((ppw,), jnp.float32),
            pltpu.SemaphoreType.DMA,
        ],
    )
    def k(prob_hbm, m_hbm, idx_hbm, alo_hbm, aup_hbm, out_hbm,
          idx_v, m_v, alo, aup, lo, up, out_v, sem):
        wid = lax.axis_index("s") * 2 + lax.axis_index("c")
        base = wid * ppw
        pltpu.sync_copy(alo_hbm.at[pl.ds(wid * rows, rows)], alo)
        pltpu.sync_copy(aup_hbm.at[pl.ds(wid * rows, rows)], aup)

        handles = []
        for r in range(rows):
            handles.append(pltpu.async_copy(prob_hbm.at[alo.at[r]], lo.at[r], sem))
            handles.append(pltpu.async_copy(prob_hbm.at[aup.at[r]], up.at[r], sem))
        pltpu.sync_copy(idx_hbm.at[pl.ds(base, ppw)], idx_v)
        pltpu.sync_copy(m_hbm.at[pl.ds(base, ppw)], m_v)
        for h in handles:
            h.wait()

        def comb_body(j, carry):
            r = j // 8
            cc = (j % 8) * 16
            v = idx_v[pl.ds(j * 16, 16)]
            mm = m_v[pl.ds(j * 16, 16)]
            l = jnp.where(v > 0, lo[r, pl.ds(cc, 16)], 0.0)
            u = jnp.where(v < cls - 1, up[r, pl.ds(cc, 16)], 0.0)
            g = jnp.maximum(l, u)
            vf = v.astype(jnp.float32)
            nbr = jnp.where(u > l, vf + 1.0, vf - 1.0)
            fl = (mm * vf + g * nbr) / (mm + g)
            out_v[pl.ds(j * 16, 16)] = fl * jnp.float32(0.035) - jnp.float32(4.0)
            return carry

        lax.fori_loop(0, ppw // 16, comb_body, 0)
        pltpu.sync_copy(out_v, out_hbm.at[pl.ds(base, ppw)])

    return k(prob_flat, m_flat, idx_flat, alo_flat, aup_flat)


def kernel(prob):
    hei, wid, cls = prob.shape
    npix = hei * wid
    m, idx, alo, aup = _tc_stage(prob)
    flat = (
        prob.reshape(npix // 8, 8, cls // 128, 128)
        .transpose(0, 2, 1, 3)
        .reshape(-1)
    )
    disp = _sc_stage(flat, m.reshape(-1), idx.reshape(-1),
                     alo.reshape(npix // 128, 128), aup.reshape(npix // 128, 128),
                     cls)
    return disp.reshape(hei, wid)


# speedup vs baseline: 1.1140x; 1.0304x over previous
import functools

import jax
import jax.numpy as jnp
from jax import lax
from jax.experimental import pallas as pl
from jax.experimental.pallas import tpu as pltpu
from jax.experimental.pallas import tpu_sc as plsc


_BH = 32


def _tc_kernel(prob_ref, m_ref, idx_ref):
    x = prob_ref[...]
    xt = jnp.swapaxes(x, 1, 2)
    c = xt.shape[1]
    m = jnp.max(xt, axis=1)
    iota = lax.broadcasted_iota(jnp.int32, xt.shape, 1)
    idx = jnp.min(jnp.where(xt == m[:, None, :], iota, c), axis=1)
    m_ref[...] = m
    idx_ref[...] = idx


def _tc_stage(prob):
    hei, wid, cls = prob.shape
    grid = hei // _BH
    pix_spec = pl.BlockSpec((_BH, wid), lambda i: (i, 0))
    return pl.pallas_call(
        _tc_kernel,
        grid=(grid,),
        in_specs=[pl.BlockSpec((_BH, wid, cls), lambda i: (i, 0, 0))],
        out_specs=(pix_spec, pix_spec),
        out_shape=(
            jax.ShapeDtypeStruct((hei, wid), jnp.float32),
            jax.ShapeDtypeStruct((hei, wid), jnp.int32),
        ),
    )(prob)


def _sc_stage(prob_flat, m_flat, idx_flat, cls):
    n = m_flat.shape[0]
    nw = 32
    ppw = n // nw
    rows = ppw // 128
    mesh = plsc.VectorSubcoreMesh(core_axis_name="c", subcore_axis_name="s")

    @functools.partial(
        pl.kernel,
        mesh=mesh,
        out_type=jax.ShapeDtypeStruct((n,), jnp.float32),
        scratch_types=[
            pltpu.VMEM((ppw,), jnp.int32),
            pltpu.VMEM((ppw,), jnp.float32),
            pltpu.VMEM((rows, 128), jnp.int32),
            pltpu.VMEM((rows, 128), jnp.int32),
            pltpu.VMEM((rows, 128), jnp.float32),
            pltpu.VMEM((rows, 128), jnp.float32),
            pltpu.VMEM((ppw,), jnp.float32),
            pltpu.SemaphoreType.DMA,
        ],
    )
    def k(prob_hbm, m_hbm, idx_hbm, out_hbm,
          idx_v, m_v, alo, aup, lo, up, out_v, sem):
        wid = lax.axis_index("s") * 2 + lax.axis_index("c")
        base = wid * ppw
        pltpu.sync_copy(idx_hbm.at[pl.ds(base, ppw)], idx_v)
        lane = lax.iota(jnp.int32, 16)

        def tiled_addr(p, c):
            t = (p >> 3) * (cls // 128) + (c >> 7)
            return t * 1024 + (p & 7) * 128 + (c & 127)

        def addr_body(j, carry):
            r = j // 4
            cc = (j % 4) * 32
            for s in range(2):
                o = j * 32 + s * 16
                v = idx_v[pl.ds(o, 16)]
                p = base + o + lane
                alo[r, pl.ds(cc + s * 16, 16)] = tiled_addr(p, jnp.maximum(v - 1, 0))
                aup[r, pl.ds(cc + s * 16, 16)] = tiled_addr(p, jnp.minimum(v + 1, cls - 1))
            return carry

        lax.fori_loop(0, ppw // 32, addr_body, 0)

        handles = []
        for r in range(rows):
            handles.append(pltpu.async_copy(prob_hbm.at[alo.at[r]], lo.at[r], sem))
            handles.append(pltpu.async_copy(prob_hbm.at[aup.at[r]], up.at[r], sem))
        pltpu.sync_copy(m_hbm.at[pl.ds(base, ppw)], m_v)
        for h in handles:
            h.wait()

        def comb_body(j, carry):
            r = j // 4
            cc = (j % 4) * 32
            for s in range(2):
                o = j * 32 + s * 16
                v = idx_v[pl.ds(o, 16)]
                mm = m_v[pl.ds(o, 16)]
                l = jnp.where(v > 0, lo[r, pl.ds(cc + s * 16, 16)], 0.0)
                u = jnp.where(v < cls - 1, up[r, pl.ds(cc + s * 16, 16)], 0.0)
                g = jnp.maximum(l, u)
                vf = v.astype(jnp.float32)
                nbr = jnp.where(u > l, vf + 1.0, vf - 1.0)
                fl = (mm * vf + g * nbr) / (mm + g)
                out_v[pl.ds(o, 16)] = fl * jnp.float32(0.035) - jnp.float32(4.0)
            return carry

        lax.fori_loop(0, ppw // 32, comb_body, 0)
        pltpu.sync_copy(out_v, out_hbm.at[pl.ds(base, ppw)])

    return k(prob_flat, m_flat, idx_flat)


def kernel(prob):
    hei, wid, cls = prob.shape
    npix = hei * wid
    m, idx = _tc_stage(prob)
    flat = (
        prob.reshape(npix // 8, 8, cls // 128, 128)
        .transpose(0, 2, 1, 3)
        .reshape(-1)
    )
    disp = _sc_stage(flat, m.reshape(-1), idx.reshape(-1), cls)
    return disp.reshape(hei, wid)


# speedup vs baseline: 1.1146x; 1.0006x over previous
import functools

import jax
import jax.numpy as jnp
from jax import lax
from jax.experimental import pallas as pl
from jax.experimental.pallas import tpu as pltpu
from jax.experimental.pallas import tpu_sc as plsc


_BH = 32


def _tc_kernel(prob_ref, m_ref, idx_ref):
    x = prob_ref[...]
    xt = jnp.swapaxes(x, 1, 2)
    c = xt.shape[1]
    m = jnp.max(xt, axis=1)
    iota = lax.broadcasted_iota(jnp.int32, xt.shape, 1)
    idx = jnp.min(jnp.where(xt == m[:, None, :], iota, c), axis=1)
    m_ref[...] = m
    idx_ref[...] = idx


def _tc_stage(prob):
    hei, wid, cls = prob.shape
    grid = hei // _BH
    pix_spec = pl.BlockSpec((_BH, wid), lambda i: (i, 0))
    return pl.pallas_call(
        _tc_kernel,
        grid=(grid,),
        in_specs=[pl.BlockSpec((_BH, wid, cls), lambda i: (i, 0, 0))],
        out_specs=(pix_spec, pix_spec),
        out_shape=(
            jax.ShapeDtypeStruct((hei, wid), jnp.float32),
            jax.ShapeDtypeStruct((hei, wid), jnp.int32),
        ),
    )(prob)


def _sc_stage(prob_flat, m_flat, idx_flat, cls):
    n = m_flat.shape[0]
    nw = 32
    ppw = n // nw
    rows = ppw // 128
    mesh = plsc.VectorSubcoreMesh(core_axis_name="c", subcore_axis_name="s")

    @functools.partial(
        pl.kernel,
        mesh=mesh,
        out_type=jax.ShapeDtypeStruct((n,), jnp.float32),
        scratch_types=[
            pltpu.VMEM((ppw,), jnp.int32),
            pltpu.VMEM((ppw,), jnp.float32),
            pltpu.VMEM((rows, 128), jnp.int32),
            pltpu.VMEM((rows, 128), jnp.int32),
            pltpu.VMEM((rows, 128), jnp.float32),
            pltpu.VMEM((rows, 128), jnp.float32),
            pltpu.VMEM((ppw,), jnp.float32),
            pltpu.SemaphoreType.DMA,
            pltpu.SemaphoreType.DMA,
            pltpu.SemaphoreType.DMA,
            pltpu.SemaphoreType.DMA,
        ],
    )
    def k(prob_hbm, m_hbm, idx_hbm, out_hbm,
          idx_v, m_v, alo, aup, lo, up, out_v, sem0, sem1, sem2, sem3):
        sems = [sem0, sem1, sem2, sem3]
        wid = lax.axis_index("s") * 2 + lax.axis_index("c")
        base = wid * ppw
        pltpu.sync_copy(idx_hbm.at[pl.ds(base, ppw)], idx_v)
        lane = lax.iota(jnp.int32, 16)

        def tiled_addr(p, c):
            t = (p >> 3) * (cls // 128) + (c >> 7)
            return t * 1024 + (p & 7) * 128 + (c & 127)

        def addr_body(j, carry):
            r = j // 4
            cc = (j % 4) * 32
            for s in range(2):
                o = j * 32 + s * 16
                v = idx_v[pl.ds(o, 16)]
                p = base + o + lane
                alo[r, pl.ds(cc + s * 16, 16)] = tiled_addr(p, jnp.maximum(v - 1, 0))
                aup[r, pl.ds(cc + s * 16, 16)] = tiled_addr(p, jnp.minimum(v + 1, cls - 1))
            return carry

        lax.fori_loop(0, ppw // 32, addr_body, 0)

        nwave = 4
        wrows = rows // nwave
        handles = []
        for w in range(nwave):
            for r in range(w * wrows, (w + 1) * wrows):
                handles.append(pltpu.async_copy(prob_hbm.at[alo.at[r]], lo.at[r], sems[w]))
                handles.append(pltpu.async_copy(prob_hbm.at[aup.at[r]], up.at[r], sems[w]))
        pltpu.sync_copy(m_hbm.at[pl.ds(base, ppw)], m_v)

        def comb_body(j, carry):
            r = j // 4
            cc = (j % 4) * 32
            for s in range(2):
                o = j * 32 + s * 16
                v = idx_v[pl.ds(o, 16)]
                mm = m_v[pl.ds(o, 16)]
                l = jnp.where(v > 0, lo[r, pl.ds(cc + s * 16, 16)], 0.0)
                u = jnp.where(v < cls - 1, up[r, pl.ds(cc + s * 16, 16)], 0.0)
                g = jnp.maximum(l, u)
                vf = v.astype(jnp.float32)
                nbr = jnp.where(u > l, vf + 1.0, vf - 1.0)
                fl = (mm * vf + g * nbr) / (mm + g)
                out_v[pl.ds(o, 16)] = fl * jnp.float32(0.035) - jnp.float32(4.0)
            return carry

        jpw = (ppw // 32) // nwave
        for w in range(nwave):
            for h in handles[w * 2 * wrows:(w + 1) * 2 * wrows]:
                h.wait()
            lax.fori_loop(w * jpw, (w + 1) * jpw, comb_body, 0)
        pltpu.sync_copy(out_v, out_hbm.at[pl.ds(base, ppw)])

    return k(prob_flat, m_flat, idx_flat)


def kernel(prob):
    hei, wid, cls = prob.shape
    npix = hei * wid
    m, idx = _tc_stage(prob)
    flat = (
        prob.reshape(npix // 8, 8, cls // 128, 128)
        .transpose(0, 2, 1, 3)
        .reshape(-1)
    )
    disp = _sc_stage(flat, m.reshape(-1), idx.reshape(-1), cls)
    return disp.reshape(hei, wid)
